# Initial kernel scaffold; baseline (speedup 1.0000x reference)
import jax
import jax.numpy as jnp
from jax.experimental import pallas as pl


def kernel(datapoints, centroid):
    raise NotImplementedError("write your pallas kernel here")


# speedup vs baseline: 1.1232x; 1.1232x over previous
import functools

import jax
import jax.numpy as jnp
from jax import lax
from jax.experimental import pallas as pl
from jax.experimental.pallas import tpu as pltpu
from jax.experimental.pallas import tpu_sc as plsc

_B = 4096
_K = 1024
_D = 256
_SMOOTH = 10.0

_BM = 512
_GB = _B // _BM

_NC = 2
_NS = 16
_NW = _NC * _NS
_PTS = _B // _NW


def _pass_a_body(dp_ref, cent_ref, cos_ref, idx_ref, best_ref, dist_ref):
    dp = dp_ref[...]
    cent = cent_ref[...]
    num = lax.dot_general(dp, cent, (((1,), (1,)), ((), ())),
                          preferred_element_type=jnp.float32,
                          precision=lax.Precision.HIGHEST)
    dn2 = jnp.sum(dp * dp, axis=1, keepdims=True)
    cn2 = jnp.sum(cent * cent, axis=1)
    denom = jnp.maximum(jnp.sqrt(dn2) * jnp.sqrt(cn2)[None, :], 1e-8)
    cos = num / denom
    cos_ref[...] = cos
    row_max = jnp.max(cos, axis=1, keepdims=True)
    iota = lax.broadcasted_iota(jnp.int32, (_BM, _K), 1)
    idx = jnp.min(jnp.where(cos == row_max, iota, _K), axis=1)
    onehot = iota == idx[:, None]
    num_at = jnp.sum(jnp.where(onehot, num, 0.0), axis=1)
    cn2_at = jnp.sum(jnp.where(onehot, cn2[None, :], 0.0), axis=1)
    sq = dn2[:, 0] + cn2_at - 2.0 * num_at
    dist = jnp.sqrt(jnp.maximum(sq, 1e-12))
    idx_ref[0, 0, :] = idx
    best_ref[0, 0, :] = row_max[:, 0]
    dist_ref[0, 0, :] = dist


def _pass_a(dp, cent):
    return pl.pallas_call(
        _pass_a_body,
        grid=(_GB,),
        in_specs=[
            pl.BlockSpec((_BM, _D), lambda i: (i, 0)),
            pl.BlockSpec((_K, _D), lambda i: (0, 0)),
        ],
        out_specs=[
            pl.BlockSpec((_BM, _K), lambda i: (i, 0)),
            pl.BlockSpec((1, 1, _BM), lambda i: (i, 0, 0)),
            pl.BlockSpec((1, 1, _BM), lambda i: (i, 0, 0)),
            pl.BlockSpec((1, 1, _BM), lambda i: (i, 0, 0)),
        ],
        out_shape=[
            jax.ShapeDtypeStruct((_B, _K), jnp.float32),
            jax.ShapeDtypeStruct((_GB, 1, _BM), jnp.int32),
            jax.ShapeDtypeStruct((_GB, 1, _BM), jnp.float32),
            jax.ShapeDtypeStruct((_GB, 1, _BM), jnp.float32),
        ],
    )(dp, cent)


def _sc_hist(idx, dist):
    mesh = plsc.VectorSubcoreMesh(core_axis_name="c", subcore_axis_name="s")

    @functools.partial(
        pl.kernel, mesh=mesh,
        out_type=jax.ShapeDtypeStruct((_NC, 2, _K), jnp.float32),
        scratch_types=[
            pltpu.VMEM((_PTS,), jnp.int32),
            pltpu.VMEM((_PTS,), jnp.float32),
            pltpu.VMEM((_PTS,), jnp.float32),
            pltpu.VMEM((_K,), jnp.float32),
            pltpu.VMEM_SHARED((_K,), jnp.float32),
            pltpu.VMEM_SHARED((_K,), jnp.float32),
        ],
    )
    def k(idx_hbm, dist_hbm, out_hbm, idx_v, dist_v, ones_v, zer_v,
          sh_cnt, sh_sum):
        c = lax.axis_index("c")
        s = lax.axis_index("s")
        wid = c * _NS + s
        base = wid * _PTS
        pltpu.sync_copy(idx_hbm.at[pl.ds(base, _PTS)], idx_v)
        pltpu.sync_copy(dist_hbm.at[pl.ds(base, _PTS)], dist_v)
        for i in range(_PTS // 16):
            ones_v[pl.ds(i * 16, 16)] = jnp.ones((16,), jnp.float32)

        @pl.when(s == 0)
        def _zero():
            for i in range(_K // 16):
                zer_v[pl.ds(i * 16, 16)] = jnp.zeros((16,), jnp.float32)
            pltpu.sync_copy(zer_v, sh_cnt)
            pltpu.sync_copy(zer_v, sh_sum)

        plsc.subcore_barrier()
        pltpu.sync_copy(ones_v, sh_cnt.at[idx_v], add=True)
        pltpu.sync_copy(dist_v, sh_sum.at[idx_v], add=True)
        plsc.subcore_barrier()

        @pl.when(s == 0)
        def _emit():
            pltpu.sync_copy(sh_cnt, out_hbm.at[c, 0])
            pltpu.sync_copy(sh_sum, out_hbm.at[c, 1])

    return k(idx, dist)


def _pass_c_body(cos_ref, idx_ref, best_ref, part_ref, out_ref):
    i = pl.program_id(0)
    cnt = part_ref[0, 0, :] + part_ref[1, 0, :]
    l2 = part_ref[0, 1, :] + part_ref[1, 1, :]
    scnt = jnp.where(cnt == 0.0, 1.0, cnt)
    phi = (l2 / scnt) / jnp.log(scnt + _SMOOTH)
    zmask = jnp.where(cnt == 0.0, -10000.0, 0.0)
    cos = cos_ref[...]
    smooth = zmask[None, :] + cos / (phi[None, :] + zmask[None, :])
    m = jnp.max(smooth, axis=1, keepdims=True)
    logz = m[:, 0] + jnp.log(jnp.sum(jnp.exp(smooth - m), axis=1))
    idx = idx_ref[0, 0, :]
    iota = lax.broadcasted_iota(jnp.int32, (_BM, _K), 1)
    onehot = iota == idx[:, None]
    phi_at = jnp.sum(jnp.where(onehot, phi[None, :], 0.0), axis=1)
    picked = best_ref[0, 0, :] / phi_at
    part = jnp.sum(logz - picked)

    @pl.when(i == 0)
    def _init():
        out_ref[0, 0] = 0.0

    out_ref[0, 0] += part / _B


def _pass_c(cos, idx3, best3, partials):
    return pl.pallas_call(
        _pass_c_body,
        grid=(_GB,),
        in_specs=[
            pl.BlockSpec((_BM, _K), lambda i: (i, 0)),
            pl.BlockSpec((1, 1, _BM), lambda i: (i, 0, 0)),
            pl.BlockSpec((1, 1, _BM), lambda i: (i, 0, 0)),
            pl.BlockSpec((_NC, 2, _K), lambda i: (0, 0, 0)),
        ],
        out_specs=pl.BlockSpec(memory_space=pltpu.SMEM),
        out_shape=jax.ShapeDtypeStruct((1, 1), jnp.float32),
    )(cos, idx3, best3, partials)


def kernel(datapoints, centroid):
    cos, idx3, best3, dist3 = _pass_a(datapoints, centroid)
    idx = idx3.reshape(_B)
    dist = dist3.reshape(_B)
    partials = _sc_hist(idx, dist)
    loss = _pass_c(cos, idx3, best3, partials)
    return loss.reshape(())


# speedup vs baseline: 1.1786x; 1.0493x over previous
import functools

import jax
import jax.numpy as jnp
from jax import lax
from jax.experimental import pallas as pl
from jax.experimental.pallas import tpu as pltpu
from jax.experimental.pallas import tpu_sc as plsc

_B = 4096
_K = 1024
_D = 256
_SMOOTH = 10.0

_BM = 512
_GB = _B // _BM

_NC = 2
_NS = 16
_NW = _NC * _NS
_PTS = _B // _NW


def _pass_a_body(dp_ref, cent_ref, cos_ref, idx_ref, best_ref, dist_ref):
    dp = dp_ref[...]
    cent = cent_ref[...]
    num = lax.dot_general(dp, cent, (((1,), (1,)), ((), ())),
                          preferred_element_type=jnp.float32,
                          precision=lax.Precision.HIGHEST)
    dn2 = jnp.sum(dp * dp, axis=1, keepdims=True)
    cn2 = jnp.sum(cent * cent, axis=1)
    inv_dn = 1.0 / jnp.maximum(jnp.sqrt(dn2), 1e-4)
    cn = jnp.maximum(jnp.sqrt(cn2), 1e-4)
    inv_cn = 1.0 / cn
    scaled = num * inv_cn[None, :]
    ms = jnp.max(scaled, axis=1, keepdims=True)
    iota = lax.broadcasted_iota(jnp.int32, (_BM, _K), 1)
    idx = jnp.min(jnp.where(scaled == ms, iota, _K), axis=1)
    onehot = iota == idx[:, None]
    cn2_at = jnp.sum(jnp.where(onehot, cn2[None, :], 0.0), axis=1)
    cos_ref[...] = scaled * inv_dn
    num_at = ms[:, 0] * jnp.maximum(jnp.sqrt(cn2_at), 1e-4)
    sq = dn2[:, 0] + cn2_at - 2.0 * num_at
    dist = jnp.sqrt(jnp.maximum(sq, 1e-12))
    idx_ref[0, 0, :] = idx
    best_ref[0, 0, :] = ms[:, 0] * inv_dn[:, 0]
    dist_ref[0, 0, :] = dist


def _pass_a(dp, cent):
    return pl.pallas_call(
        _pass_a_body,
        grid=(_GB,),
        in_specs=[
            pl.BlockSpec((_BM, _D), lambda i: (i, 0)),
            pl.BlockSpec((_K, _D), lambda i: (0, 0)),
        ],
        out_specs=[
            pl.BlockSpec((_BM, _K), lambda i: (i, 0)),
            pl.BlockSpec((1, 1, _BM), lambda i: (i, 0, 0)),
            pl.BlockSpec((1, 1, _BM), lambda i: (i, 0, 0)),
            pl.BlockSpec((1, 1, _BM), lambda i: (i, 0, 0)),
        ],
        out_shape=[
            jax.ShapeDtypeStruct((_B, _K), jnp.float32),
            jax.ShapeDtypeStruct((_GB, 1, _BM), jnp.int32),
            jax.ShapeDtypeStruct((_GB, 1, _BM), jnp.float32),
            jax.ShapeDtypeStruct((_GB, 1, _BM), jnp.float32),
        ],
    )(dp, cent)


def _sc_hist(idx, dist, best):
    mesh = plsc.VectorSubcoreMesh(core_axis_name="c", subcore_axis_name="s")

    @functools.partial(
        pl.kernel, mesh=mesh,
        out_type=jax.ShapeDtypeStruct((_NC, 4, _K), jnp.float32),
        scratch_types=[
            pltpu.VMEM((_PTS,), jnp.int32),
            pltpu.VMEM((_PTS,), jnp.float32),
            pltpu.VMEM((_PTS,), jnp.float32),
            pltpu.VMEM((_PTS,), jnp.float32),
            pltpu.VMEM((_K,), jnp.float32),
            pltpu.VMEM_SHARED((_K,), jnp.float32),
            pltpu.VMEM_SHARED((_K,), jnp.float32),
            pltpu.VMEM_SHARED((_K,), jnp.float32),
        ],
    )
    def k(idx_hbm, dist_hbm, best_hbm, out_hbm, idx_v, dist_v, best_v,
          ones_v, zer_v, sh_cnt, sh_sum, sh_best):
        c = lax.axis_index("c")
        s = lax.axis_index("s")
        wid = c * _NS + s
        base = wid * _PTS
        pltpu.sync_copy(idx_hbm.at[pl.ds(base, _PTS)], idx_v)
        pltpu.sync_copy(dist_hbm.at[pl.ds(base, _PTS)], dist_v)
        pltpu.sync_copy(best_hbm.at[pl.ds(base, _PTS)], best_v)
        for i in range(_PTS // 16):
            ones_v[pl.ds(i * 16, 16)] = jnp.ones((16,), jnp.float32)

        @pl.when(s == 0)
        def _zero():
            for i in range(_K // 16):
                zer_v[pl.ds(i * 16, 16)] = jnp.zeros((16,), jnp.float32)
            pltpu.sync_copy(zer_v, sh_cnt)
            pltpu.sync_copy(zer_v, sh_sum)
            pltpu.sync_copy(zer_v, sh_best)
            pltpu.sync_copy(zer_v, out_hbm.at[c, 3])

        plsc.subcore_barrier()
        pltpu.sync_copy(ones_v, sh_cnt.at[idx_v], add=True)
        pltpu.sync_copy(dist_v, sh_sum.at[idx_v], add=True)
        pltpu.sync_copy(best_v, sh_best.at[idx_v], add=True)
        plsc.subcore_barrier()

        @pl.when(s == 0)
        def _emit():
            pltpu.sync_copy(sh_cnt, out_hbm.at[c, 0])
            pltpu.sync_copy(sh_sum, out_hbm.at[c, 1])
            pltpu.sync_copy(sh_best, out_hbm.at[c, 2])

    return k(idx, dist, best)


def _pass_c_body(cos_ref, part_ref, out_ref):
    i = pl.program_id(0)
    cnt = part_ref[0, 0, :] + part_ref[1, 0, :]
    l2 = part_ref[0, 1, :] + part_ref[1, 1, :]
    bs = part_ref[0, 2, :] + part_ref[1, 2, :]
    scnt = jnp.where(cnt == 0.0, 1.0, cnt)
    phi = (l2 / scnt) / jnp.log(scnt + _SMOOTH)
    zmask = jnp.where(cnt == 0.0, -10000.0, 0.0)
    col_scale = 1.0 / (phi + zmask)
    smooth = zmask[None, :] + cos_ref[...] * col_scale[None, :]
    m = jnp.max(smooth, axis=1, keepdims=True)
    logz = m[:, 0] + jnp.log(jnp.sum(jnp.exp(smooth - m), axis=1))
    part = jnp.sum(logz)

    @pl.when(i == 0)
    def _init():
        picked_sum = jnp.sum(jnp.where(cnt == 0.0, 0.0, bs / phi))
        out_ref[0, 0] = -picked_sum / _B

    out_ref[0, 0] += part / _B


def _pass_c(cos, partials):
    return pl.pallas_call(
        _pass_c_body,
        grid=(_GB,),
        in_specs=[
            pl.BlockSpec((_BM, _K), lambda i: (i, 0)),
            pl.BlockSpec((_NC, 4, _K), lambda i: (0, 0, 0)),
        ],
        out_specs=pl.BlockSpec(memory_space=pltpu.SMEM),
        out_shape=jax.ShapeDtypeStruct((1, 1), jnp.float32),
    )(cos, partials)


def kernel(datapoints, centroid):
    cos, idx3, best3, dist3 = _pass_a(datapoints, centroid)
    idx = idx3.reshape(_B)
    dist = dist3.reshape(_B)
    best = best3.reshape(_B)
    partials = _sc_hist(idx, dist, best)
    loss = _pass_c(cos, partials)
    return loss.reshape(())


# speedup vs baseline: 1.3912x; 1.1804x over previous
import functools

import jax
import jax.numpy as jnp
from jax import lax
from jax.experimental import pallas as pl
from jax.experimental.pallas import tpu as pltpu
from jax.experimental.pallas import tpu_sc as plsc

_B = 4096
_K = 1024
_D = 256
_SMOOTH = 10.0

_BM = 512
_GB = _B // _BM

_NC = 2
_NS = 16
_NW = _NC * _NS
_PTS = _B // _NW


def _pass_a_body(dp_ref, cent_ref, cos_ref, idx_ref, best_ref, dist_ref):
    dp = dp_ref[...]
    cent = cent_ref[...]
    num = lax.dot_general(dp, cent, (((1,), (1,)), ((), ())),
                          preferred_element_type=jnp.float32,
                          precision=lax.Precision.DEFAULT)
    dn2 = jnp.sum(dp * dp, axis=1, keepdims=True)
    cn2 = jnp.sum(cent * cent, axis=1)
    inv_dn = 1.0 / jnp.maximum(jnp.sqrt(dn2), 1e-4)
    cn = jnp.maximum(jnp.sqrt(cn2), 1e-4)
    inv_cn = 1.0 / cn
    scaled = num * inv_cn[None, :]
    ms = jnp.max(scaled, axis=1, keepdims=True)
    iota = lax.broadcasted_iota(jnp.int32, (_BM, _K), 1)
    idx = jnp.min(jnp.where(scaled == ms, iota, _K), axis=1)
    onehot = iota == idx[:, None]
    cn2_at = jnp.sum(jnp.where(onehot, cn2[None, :], 0.0), axis=1)
    cos_ref[...] = scaled * inv_dn
    num_at = ms[:, 0] * jnp.maximum(jnp.sqrt(cn2_at), 1e-4)
    sq = dn2[:, 0] + cn2_at - 2.0 * num_at
    dist = jnp.sqrt(jnp.maximum(sq, 1e-12))
    idx_ref[0, 0, :] = idx
    best_ref[0, 0, :] = ms[:, 0] * inv_dn[:, 0]
    dist_ref[0, 0, :] = dist


def _pass_a(dp, cent):
    return pl.pallas_call(
        _pass_a_body,
        grid=(_GB,),
        in_specs=[
            pl.BlockSpec((_BM, _D), lambda i: (i, 0)),
            pl.BlockSpec((_K, _D), lambda i: (0, 0)),
        ],
        out_specs=[
            pl.BlockSpec((_BM, _K), lambda i: (i, 0)),
            pl.BlockSpec((1, 1, _BM), lambda i: (i, 0, 0)),
            pl.BlockSpec((1, 1, _BM), lambda i: (i, 0, 0)),
            pl.BlockSpec((1, 1, _BM), lambda i: (i, 0, 0)),
        ],
        out_shape=[
            jax.ShapeDtypeStruct((_B, _K), jnp.float32),
            jax.ShapeDtypeStruct((_GB, 1, _BM), jnp.int32),
            jax.ShapeDtypeStruct((_GB, 1, _BM), jnp.float32),
            jax.ShapeDtypeStruct((_GB, 1, _BM), jnp.float32),
        ],
    )(dp, cent)


def _sc_hist(idx, dist, best):
    mesh = plsc.VectorSubcoreMesh(core_axis_name="c", subcore_axis_name="s")

    @functools.partial(
        pl.kernel, mesh=mesh,
        out_type=jax.ShapeDtypeStruct((_NC, 4, _K), jnp.float32),
        scratch_types=[
            pltpu.VMEM((_PTS,), jnp.int32),
            pltpu.VMEM((_PTS,), jnp.float32),
            pltpu.VMEM((_PTS,), jnp.float32),
            pltpu.VMEM((_PTS,), jnp.float32),
            pltpu.VMEM((_K,), jnp.float32),
            pltpu.VMEM_SHARED((_K,), jnp.float32),
            pltpu.VMEM_SHARED((_K,), jnp.float32),
            pltpu.VMEM_SHARED((_K,), jnp.float32),
        ],
    )
    def k(idx_hbm, dist_hbm, best_hbm, out_hbm, idx_v, dist_v, best_v,
          ones_v, zer_v, sh_cnt, sh_sum, sh_best):
        c = lax.axis_index("c")
        s = lax.axis_index("s")
        wid = c * _NS + s
        base = wid * _PTS
        pltpu.sync_copy(idx_hbm.at[pl.ds(base, _PTS)], idx_v)
        pltpu.sync_copy(dist_hbm.at[pl.ds(base, _PTS)], dist_v)
        pltpu.sync_copy(best_hbm.at[pl.ds(base, _PTS)], best_v)
        for i in range(_PTS // 16):
            ones_v[pl.ds(i * 16, 16)] = jnp.ones((16,), jnp.float32)

        @pl.when(s == 0)
        def _zero():
            for i in range(_K // 16):
                zer_v[pl.ds(i * 16, 16)] = jnp.zeros((16,), jnp.float32)
            pltpu.sync_copy(zer_v, sh_cnt)
            pltpu.sync_copy(zer_v, sh_sum)
            pltpu.sync_copy(zer_v, sh_best)
            pltpu.sync_copy(zer_v, out_hbm.at[c, 3])

        plsc.subcore_barrier()
        pltpu.sync_copy(ones_v, sh_cnt.at[idx_v], add=True)
        pltpu.sync_copy(dist_v, sh_sum.at[idx_v], add=True)
        pltpu.sync_copy(best_v, sh_best.at[idx_v], add=True)
        plsc.subcore_barrier()

        @pl.when(s == 0)
        def _emit():
            pltpu.sync_copy(sh_cnt, out_hbm.at[c, 0])
            pltpu.sync_copy(sh_sum, out_hbm.at[c, 1])
            pltpu.sync_copy(sh_best, out_hbm.at[c, 2])

    return k(idx, dist, best)


def _pass_c_body(cos_ref, part_ref, out_ref):
    i = pl.program_id(0)
    cnt = part_ref[0, 0, :] + part_ref[1, 0, :]
    l2 = part_ref[0, 1, :] + part_ref[1, 1, :]
    bs = part_ref[0, 2, :] + part_ref[1, 2, :]
    scnt = jnp.where(cnt == 0.0, 1.0, cnt)
    phi = (l2 / scnt) / jnp.log(scnt + _SMOOTH)
    zmask = jnp.where(cnt == 0.0, -10000.0, 0.0)
    col_scale = 1.0 / (phi + zmask)
    smooth = zmask[None, :] + cos_ref[...] * col_scale[None, :]
    m = jnp.max(smooth, axis=1, keepdims=True)
    logz = m[:, 0] + jnp.log(jnp.sum(jnp.exp(smooth - m), axis=1))
    part = jnp.sum(logz)

    @pl.when(i == 0)
    def _init():
        picked_sum = jnp.sum(jnp.where(cnt == 0.0, 0.0, bs / phi))
        out_ref[0, 0] = -picked_sum / _B

    out_ref[0, 0] += part / _B


def _pass_c(cos, partials):
    return pl.pallas_call(
        _pass_c_body,
        grid=(_GB,),
        in_specs=[
            pl.BlockSpec((_BM, _K), lambda i: (i, 0)),
            pl.BlockSpec((_NC, 4, _K), lambda i: (0, 0, 0)),
        ],
        out_specs=pl.BlockSpec(memory_space=pltpu.SMEM),
        out_shape=jax.ShapeDtypeStruct((1, 1), jnp.float32),
    )(cos, partials)


def kernel(datapoints, centroid):
    cos, idx3, best3, dist3 = _pass_a(datapoints, centroid)
    idx = idx3.reshape(_B)
    dist = dist3.reshape(_B)
    best = best3.reshape(_B)
    partials = _sc_hist(idx, dist, best)
    loss = _pass_c(cos, partials)
    return loss.reshape(())


# speedup vs baseline: 1.4242x; 1.0237x over previous
import functools

import jax
import jax.numpy as jnp
from jax import lax
from jax.experimental import pallas as pl
from jax.experimental.pallas import tpu as pltpu
from jax.experimental.pallas import tpu_sc as plsc

_B = 4096
_K = 1024
_D = 256
_SMOOTH = 10.0

_BM = 512
_GB = _B // _BM

_NC = 2
_NS = 16
_NW = _NC * _NS
_PTS = _B // _NW


def _pass_a_body(dp_ref, cent_ref, cos_ref, idx_ref, best_ref, dist_ref):
    dp = dp_ref[...]
    cent = cent_ref[...]
    num = lax.dot_general(dp, cent, (((1,), (1,)), ((), ())),
                          preferred_element_type=jnp.float32,
                          precision=lax.Precision.DEFAULT)
    dn2 = jnp.sum(dp * dp, axis=1, keepdims=True)
    cn2 = jnp.sum(cent * cent, axis=1)
    inv_dn = 1.0 / jnp.maximum(jnp.sqrt(dn2), 1e-4)
    cn = jnp.maximum(jnp.sqrt(cn2), 1e-4)
    inv_cn = 1.0 / cn
    scaled = num * inv_cn[None, :]
    ms = jnp.max(scaled, axis=1, keepdims=True)
    iota = lax.broadcasted_iota(jnp.int32, (_BM, _K), 1)
    idx = jnp.min(jnp.where(scaled == ms, iota, _K), axis=1)
    onehot = iota == idx[:, None]
    cn2_at = jnp.sum(jnp.where(onehot, cn2[None, :], 0.0), axis=1)
    cos_ref[...] = (scaled * inv_dn).astype(jnp.bfloat16)
    num_at = ms[:, 0] * jnp.maximum(jnp.sqrt(cn2_at), 1e-4)
    sq = dn2[:, 0] + cn2_at - 2.0 * num_at
    dist = jnp.sqrt(jnp.maximum(sq, 1e-12))
    idx_ref[0, 0, :] = idx
    best_ref[0, 0, :] = ms[:, 0] * inv_dn[:, 0]
    dist_ref[0, 0, :] = dist


def _pass_a(dp, cent):
    return pl.pallas_call(
        _pass_a_body,
        grid=(_GB,),
        in_specs=[
            pl.BlockSpec((_BM, _D), lambda i: (i, 0)),
            pl.BlockSpec((_K, _D), lambda i: (0, 0)),
        ],
        out_specs=[
            pl.BlockSpec((_BM, _K), lambda i: (i, 0)),
            pl.BlockSpec((1, 1, _BM), lambda i: (i, 0, 0)),
            pl.BlockSpec((1, 1, _BM), lambda i: (i, 0, 0)),
            pl.BlockSpec((1, 1, _BM), lambda i: (i, 0, 0)),
        ],
        out_shape=[
            jax.ShapeDtypeStruct((_B, _K), jnp.bfloat16),
            jax.ShapeDtypeStruct((_GB, 1, _BM), jnp.int32),
            jax.ShapeDtypeStruct((_GB, 1, _BM), jnp.float32),
            jax.ShapeDtypeStruct((_GB, 1, _BM), jnp.float32),
        ],
    )(dp, cent)


def _sc_hist(idx, dist, best):
    mesh = plsc.VectorSubcoreMesh(core_axis_name="c", subcore_axis_name="s")

    @functools.partial(
        pl.kernel, mesh=mesh,
        out_type=jax.ShapeDtypeStruct((_NC, 4, _K), jnp.float32),
        scratch_types=[
            pltpu.VMEM((_PTS,), jnp.int32),
            pltpu.VMEM((_PTS,), jnp.float32),
            pltpu.VMEM((_PTS,), jnp.float32),
            pltpu.VMEM((_PTS,), jnp.float32),
            pltpu.VMEM((_K,), jnp.float32),
            pltpu.VMEM_SHARED((_K,), jnp.float32),
            pltpu.VMEM_SHARED((_K,), jnp.float32),
            pltpu.VMEM_SHARED((_K,), jnp.float32),
        ],
    )
    def k(idx_hbm, dist_hbm, best_hbm, out_hbm, idx_v, dist_v, best_v,
          ones_v, zer_v, sh_cnt, sh_sum, sh_best):
        c = lax.axis_index("c")
        s = lax.axis_index("s")
        wid = c * _NS + s
        base = wid * _PTS
        pltpu.sync_copy(idx_hbm.at[pl.ds(base, _PTS)], idx_v)
        pltpu.sync_copy(dist_hbm.at[pl.ds(base, _PTS)], dist_v)
        pltpu.sync_copy(best_hbm.at[pl.ds(base, _PTS)], best_v)
        for i in range(_PTS // 16):
            ones_v[pl.ds(i * 16, 16)] = jnp.ones((16,), jnp.float32)

        @pl.when(s == 0)
        def _zero():
            for i in range(_K // 16):
                zer_v[pl.ds(i * 16, 16)] = jnp.zeros((16,), jnp.float32)
            pltpu.sync_copy(zer_v, sh_cnt)
            pltpu.sync_copy(zer_v, sh_sum)
            pltpu.sync_copy(zer_v, sh_best)
            pltpu.sync_copy(zer_v, out_hbm.at[c, 3])

        plsc.subcore_barrier()
        pltpu.sync_copy(ones_v, sh_cnt.at[idx_v], add=True)
        pltpu.sync_copy(dist_v, sh_sum.at[idx_v], add=True)
        pltpu.sync_copy(best_v, sh_best.at[idx_v], add=True)
        plsc.subcore_barrier()

        @pl.when(s == 0)
        def _emit():
            pltpu.sync_copy(sh_cnt, out_hbm.at[c, 0])
            pltpu.sync_copy(sh_sum, out_hbm.at[c, 1])
            pltpu.sync_copy(sh_best, out_hbm.at[c, 2])

    return k(idx, dist, best)


def _pass_c_body(cos_ref, part_ref, out_ref):
    i = pl.program_id(0)
    cnt = part_ref[0, 0, :] + part_ref[1, 0, :]
    l2 = part_ref[0, 1, :] + part_ref[1, 1, :]
    bs = part_ref[0, 2, :] + part_ref[1, 2, :]
    scnt = jnp.where(cnt == 0.0, 1.0, cnt)
    phi = (l2 / scnt) / jnp.log(scnt + _SMOOTH)
    zmask = jnp.where(cnt == 0.0, -10000.0, 0.0)
    col_scale = 1.0 / (phi + zmask)
    smooth = zmask[None, :] + cos_ref[...].astype(jnp.float32) * col_scale[None, :]
    m = jnp.max(smooth, axis=1, keepdims=True)
    logz = m[:, 0] + jnp.log(jnp.sum(jnp.exp(smooth - m), axis=1))
    part = jnp.sum(logz)

    @pl.when(i == 0)
    def _init():
        picked_sum = jnp.sum(jnp.where(cnt == 0.0, 0.0, bs / phi))
        out_ref[0, 0] = -picked_sum / _B

    out_ref[0, 0] += part / _B


def _pass_c(cos, partials):
    return pl.pallas_call(
        _pass_c_body,
        grid=(_GB,),
        in_specs=[
            pl.BlockSpec((_BM, _K), lambda i: (i, 0)),
            pl.BlockSpec((_NC, 4, _K), lambda i: (0, 0, 0)),
        ],
        out_specs=pl.BlockSpec(memory_space=pltpu.SMEM),
        out_shape=jax.ShapeDtypeStruct((1, 1), jnp.float32),
    )(cos, partials)


def kernel(datapoints, centroid):
    cos, idx3, best3, dist3 = _pass_a(datapoints, centroid)
    idx = idx3.reshape(_B)
    dist = dist3.reshape(_B)
    best = best3.reshape(_B)
    partials = _sc_hist(idx, dist, best)
    loss = _pass_c(cos, partials)
    return loss.reshape(())


# speedup vs baseline: 1.5356x; 1.0783x over previous
import functools

import jax
import jax.numpy as jnp
from jax import lax
from jax.experimental import pallas as pl
from jax.experimental.pallas import tpu as pltpu
from jax.experimental.pallas import tpu_sc as plsc

_B = 4096
_K = 1024
_D = 256
_SMOOTH = 10.0

_BM = 512
_GB = _B // _BM

_NC = 2
_NS = 16
_NW = _NC * _NS
_PTS = _B // _NW


def _pass_a_body(dp_ref, cent_ref, cos_ref, idx_ref, best_ref, dist_ref):
    dp = dp_ref[...]
    cent = cent_ref[...]
    num = lax.dot_general(dp, cent, (((1,), (1,)), ((), ())),
                          preferred_element_type=jnp.float32,
                          precision=lax.Precision.DEFAULT)
    dn2 = jnp.sum(dp * dp, axis=1, keepdims=True)
    cn2 = jnp.sum(cent * cent, axis=1)
    inv_dn = 1.0 / jnp.maximum(jnp.sqrt(dn2), 1e-4)
    cn = jnp.maximum(jnp.sqrt(cn2), 1e-4)
    inv_cn = 1.0 / cn
    scaled = num * inv_cn[None, :]
    ms = jnp.max(scaled, axis=1, keepdims=True)
    onehot = (scaled == ms).astype(jnp.bfloat16)
    kio = lax.broadcasted_iota(jnp.int32, (1, _K), 1)
    khi = (kio >> 5).astype(jnp.float32)
    klo = (kio & 31).astype(jnp.float32)
    w3 = jnp.concatenate([khi, klo, cn2[None, :]], axis=0).astype(jnp.bfloat16)
    res = lax.dot_general(onehot, w3, (((1,), (1,)), ((), ())),
                          preferred_element_type=jnp.float32)
    idx_f = res[:, 0] * 32.0 + res[:, 1]
    idx = jnp.minimum(idx_f, float(_K - 1)).astype(jnp.int32)
    cn2_at = res[:, 2]
    cos_ref[...] = (scaled * inv_dn).astype(jnp.bfloat16)
    num_at = ms[:, 0] * jnp.maximum(jnp.sqrt(cn2_at), 1e-4)
    sq = dn2[:, 0] + cn2_at - 2.0 * num_at
    dist = jnp.sqrt(jnp.maximum(sq, 1e-12))
    idx_ref[0, 0, :] = idx
    best_ref[0, 0, :] = ms[:, 0] * inv_dn[:, 0]
    dist_ref[0, 0, :] = dist


def _pass_a(dp, cent):
    return pl.pallas_call(
        _pass_a_body,
        grid=(_GB,),
        in_specs=[
            pl.BlockSpec((_BM, _D), lambda i: (i, 0)),
            pl.BlockSpec((_K, _D), lambda i: (0, 0)),
        ],
        out_specs=[
            pl.BlockSpec((_BM, _K), lambda i: (i, 0)),
            pl.BlockSpec((1, 1, _BM), lambda i: (i, 0, 0)),
            pl.BlockSpec((1, 1, _BM), lambda i: (i, 0, 0)),
            pl.BlockSpec((1, 1, _BM), lambda i: (i, 0, 0)),
        ],
        out_shape=[
            jax.ShapeDtypeStruct((_B, _K), jnp.bfloat16),
            jax.ShapeDtypeStruct((_GB, 1, _BM), jnp.int32),
            jax.ShapeDtypeStruct((_GB, 1, _BM), jnp.float32),
            jax.ShapeDtypeStruct((_GB, 1, _BM), jnp.float32),
        ],
    )(dp, cent)


def _sc_hist(idx, dist, best):
    mesh = plsc.VectorSubcoreMesh(core_axis_name="c", subcore_axis_name="s")

    @functools.partial(
        pl.kernel, mesh=mesh,
        out_type=jax.ShapeDtypeStruct((_NC, 4, _K), jnp.float32),
        scratch_types=[
            pltpu.VMEM((_PTS,), jnp.int32),
            pltpu.VMEM((_PTS,), jnp.float32),
            pltpu.VMEM((_PTS,), jnp.float32),
            pltpu.VMEM((_PTS,), jnp.float32),
            pltpu.VMEM((_K,), jnp.float32),
            pltpu.VMEM_SHARED((_K,), jnp.float32),
            pltpu.VMEM_SHARED((_K,), jnp.float32),
            pltpu.VMEM_SHARED((_K,), jnp.float32),
        ],
    )
    def k(idx_hbm, dist_hbm, best_hbm, out_hbm, idx_v, dist_v, best_v,
          ones_v, zer_v, sh_cnt, sh_sum, sh_best):
        c = lax.axis_index("c")
        s = lax.axis_index("s")
        wid = c * _NS + s
        base = wid * _PTS
        pltpu.sync_copy(idx_hbm.at[pl.ds(base, _PTS)], idx_v)
        pltpu.sync_copy(dist_hbm.at[pl.ds(base, _PTS)], dist_v)
        pltpu.sync_copy(best_hbm.at[pl.ds(base, _PTS)], best_v)
        for i in range(_PTS // 16):
            ones_v[pl.ds(i * 16, 16)] = jnp.ones((16,), jnp.float32)

        @pl.when(s == 0)
        def _zero():
            for i in range(_K // 16):
                zer_v[pl.ds(i * 16, 16)] = jnp.zeros((16,), jnp.float32)
            pltpu.sync_copy(zer_v, sh_cnt)
            pltpu.sync_copy(zer_v, sh_sum)
            pltpu.sync_copy(zer_v, sh_best)
            pltpu.sync_copy(zer_v, out_hbm.at[c, 3])

        plsc.subcore_barrier()
        pltpu.sync_copy(ones_v, sh_cnt.at[idx_v], add=True)
        pltpu.sync_copy(dist_v, sh_sum.at[idx_v], add=True)
        pltpu.sync_copy(best_v, sh_best.at[idx_v], add=True)
        plsc.subcore_barrier()

        @pl.when(s == 0)
        def _emit():
            pltpu.sync_copy(sh_cnt, out_hbm.at[c, 0])
            pltpu.sync_copy(sh_sum, out_hbm.at[c, 1])
            pltpu.sync_copy(sh_best, out_hbm.at[c, 2])

    return k(idx, dist, best)


def _pass_c_body(cos_ref, part_ref, out_ref):
    i = pl.program_id(0)
    cnt = part_ref[0, 0, :] + part_ref[1, 0, :]
    l2 = part_ref[0, 1, :] + part_ref[1, 1, :]
    bs = part_ref[0, 2, :] + part_ref[1, 2, :]
    scnt = jnp.where(cnt == 0.0, 1.0, cnt)
    phi = (l2 / scnt) / jnp.log(scnt + _SMOOTH)
    zmask = jnp.where(cnt == 0.0, -10000.0, 0.0)
    col_scale = 1.0 / (phi + zmask)
    smooth = zmask[None, :] + cos_ref[...].astype(jnp.float32) * col_scale[None, :]
    m = jnp.max(smooth, axis=1, keepdims=True)
    logz = m[:, 0] + jnp.log(jnp.sum(jnp.exp(smooth - m), axis=1))
    part = jnp.sum(logz)

    @pl.when(i == 0)
    def _init():
        picked_sum = jnp.sum(jnp.where(cnt == 0.0, 0.0, bs / phi))
        out_ref[0, 0] = -picked_sum / _B

    out_ref[0, 0] += part / _B


def _pass_c(cos, partials):
    return pl.pallas_call(
        _pass_c_body,
        grid=(_GB,),
        in_specs=[
            pl.BlockSpec((_BM, _K), lambda i: (i, 0)),
            pl.BlockSpec((_NC, 4, _K), lambda i: (0, 0, 0)),
        ],
        out_specs=pl.BlockSpec(memory_space=pltpu.SMEM),
        out_shape=jax.ShapeDtypeStruct((1, 1), jnp.float32),
    )(cos, partials)


def kernel(datapoints, centroid):
    cos, idx3, best3, dist3 = _pass_a(datapoints, centroid)
    idx = idx3.reshape(_B)
    dist = dist3.reshape(_B)
    best = best3.reshape(_B)
    partials = _sc_hist(idx, dist, best)
    loss = _pass_c(cos, partials)
    return loss.reshape(())


# speedup vs baseline: 1.6543x; 1.0773x over previous
import functools

import jax
import jax.numpy as jnp
from jax import lax
from jax.experimental import pallas as pl
from jax.experimental.pallas import tpu as pltpu
from jax.experimental.pallas import tpu_sc as plsc

_B = 4096
_K = 1024
_D = 256
_SMOOTH = 10.0

_BM = 1024
_GB = _B // _BM

_NC = 2
_NS = 16
_NW = _NC * _NS
_PTS = _B // _NW


def _pass_a_body(dp_ref, cent_ref, cos_ref, idx_ref, best_ref, dist_ref):
    dp = dp_ref[...]
    cent = cent_ref[...]
    num = lax.dot_general(dp, cent, (((1,), (1,)), ((), ())),
                          preferred_element_type=jnp.float32,
                          precision=lax.Precision.DEFAULT)
    dn2 = jnp.sum(dp * dp, axis=1, keepdims=True)
    cn2 = jnp.sum(cent * cent, axis=1)
    inv_dn = 1.0 / jnp.maximum(jnp.sqrt(dn2), 1e-4)
    cn = jnp.maximum(jnp.sqrt(cn2), 1e-4)
    inv_cn = 1.0 / cn
    scaled = num * inv_cn[None, :]
    ms = jnp.max(scaled, axis=1, keepdims=True)
    onehot = (scaled == ms).astype(jnp.bfloat16)
    kio = lax.broadcasted_iota(jnp.int32, (1, _K), 1)
    khi = (kio >> 5).astype(jnp.float32)
    klo = (kio & 31).astype(jnp.float32)
    w3 = jnp.concatenate([khi, klo, cn2[None, :]], axis=0).astype(jnp.bfloat16)
    res = lax.dot_general(onehot, w3, (((1,), (1,)), ((), ())),
                          preferred_element_type=jnp.float32)
    idx_f = res[:, 0] * 32.0 + res[:, 1]
    idx = jnp.minimum(idx_f, float(_K - 1)).astype(jnp.int32)
    cn2_at = res[:, 2]
    cos_ref[...] = (scaled * inv_dn).astype(jnp.bfloat16)
    num_at = ms[:, 0] * jnp.maximum(jnp.sqrt(cn2_at), 1e-4)
    sq = dn2[:, 0] + cn2_at - 2.0 * num_at
    dist = jnp.sqrt(jnp.maximum(sq, 1e-12))
    idx_ref[0, 0, :] = idx
    best_ref[0, 0, :] = ms[:, 0] * inv_dn[:, 0]
    dist_ref[0, 0, :] = dist


def _pass_a(dp, cent):
    return pl.pallas_call(
        _pass_a_body,
        grid=(_GB,),
        in_specs=[
            pl.BlockSpec((_BM, _D), lambda i: (i, 0)),
            pl.BlockSpec((_K, _D), lambda i: (0, 0)),
        ],
        out_specs=[
            pl.BlockSpec((_BM, _K), lambda i: (i, 0)),
            pl.BlockSpec((1, 1, _BM), lambda i: (i, 0, 0)),
            pl.BlockSpec((1, 1, _BM), lambda i: (i, 0, 0)),
            pl.BlockSpec((1, 1, _BM), lambda i: (i, 0, 0)),
        ],
        out_shape=[
            jax.ShapeDtypeStruct((_B, _K), jnp.bfloat16),
            jax.ShapeDtypeStruct((_GB, 1, _BM), jnp.int32),
            jax.ShapeDtypeStruct((_GB, 1, _BM), jnp.float32),
            jax.ShapeDtypeStruct((_GB, 1, _BM), jnp.float32),
        ],
    )(dp, cent)


def _sc_hist(idx, dist, best):
    mesh = plsc.VectorSubcoreMesh(core_axis_name="c", subcore_axis_name="s")

    @functools.partial(
        pl.kernel, mesh=mesh,
        out_type=jax.ShapeDtypeStruct((_NC, 4, _K), jnp.float32),
        scratch_types=[
            pltpu.VMEM((_PTS,), jnp.int32),
            pltpu.VMEM((_PTS,), jnp.float32),
            pltpu.VMEM((_PTS,), jnp.float32),
            pltpu.VMEM((_PTS,), jnp.float32),
            pltpu.VMEM((_K,), jnp.float32),
            pltpu.VMEM_SHARED((_K,), jnp.float32),
            pltpu.VMEM_SHARED((_K,), jnp.float32),
            pltpu.VMEM_SHARED((_K,), jnp.float32),
        ],
    )
    def k(idx_hbm, dist_hbm, best_hbm, out_hbm, idx_v, dist_v, best_v,
          ones_v, zer_v, sh_cnt, sh_sum, sh_best):
        c = lax.axis_index("c")
        s = lax.axis_index("s")
        wid = c * _NS + s
        base = wid * _PTS
        pltpu.sync_copy(idx_hbm.at[pl.ds(base, _PTS)], idx_v)
        pltpu.sync_copy(dist_hbm.at[pl.ds(base, _PTS)], dist_v)
        pltpu.sync_copy(best_hbm.at[pl.ds(base, _PTS)], best_v)
        for i in range(_PTS // 16):
            ones_v[pl.ds(i * 16, 16)] = jnp.ones((16,), jnp.float32)

        @pl.when(s == 0)
        def _zero():
            for i in range(_K // 16):
                zer_v[pl.ds(i * 16, 16)] = jnp.zeros((16,), jnp.float32)
            pltpu.sync_copy(zer_v, sh_cnt)
            pltpu.sync_copy(zer_v, sh_sum)
            pltpu.sync_copy(zer_v, sh_best)
            pltpu.sync_copy(zer_v, out_hbm.at[c, 3])

        plsc.subcore_barrier()
        pltpu.sync_copy(ones_v, sh_cnt.at[idx_v], add=True)
        pltpu.sync_copy(dist_v, sh_sum.at[idx_v], add=True)
        pltpu.sync_copy(best_v, sh_best.at[idx_v], add=True)
        plsc.subcore_barrier()

        @pl.when(s == 0)
        def _emit():
            pltpu.sync_copy(sh_cnt, out_hbm.at[c, 0])
            pltpu.sync_copy(sh_sum, out_hbm.at[c, 1])
            pltpu.sync_copy(sh_best, out_hbm.at[c, 2])

    return k(idx, dist, best)


def _pass_c_body(cos_ref, part_ref, out_ref):
    i = pl.program_id(0)
    cnt = part_ref[0, 0, :] + part_ref[1, 0, :]
    l2 = part_ref[0, 1, :] + part_ref[1, 1, :]
    bs = part_ref[0, 2, :] + part_ref[1, 2, :]
    scnt = jnp.where(cnt == 0.0, 1.0, cnt)
    phi = (l2 / scnt) / jnp.log(scnt + _SMOOTH)
    zmask = jnp.where(cnt == 0.0, -10000.0, 0.0)
    col_scale = 1.0 / (phi + zmask)
    smooth = zmask[None, :] + cos_ref[...].astype(jnp.float32) * col_scale[None, :]
    m = jnp.max(smooth, axis=1, keepdims=True)
    logz = m[:, 0] + jnp.log(jnp.sum(jnp.exp(smooth - m), axis=1))
    part = jnp.sum(logz)

    @pl.when(i == 0)
    def _init():
        picked_sum = jnp.sum(jnp.where(cnt == 0.0, 0.0, bs / phi))
        out_ref[0, 0] = -picked_sum / _B

    out_ref[0, 0] += part / _B


def _pass_c(cos, partials):
    return pl.pallas_call(
        _pass_c_body,
        grid=(_GB,),
        in_specs=[
            pl.BlockSpec((_BM, _K), lambda i: (i, 0)),
            pl.BlockSpec((_NC, 4, _K), lambda i: (0, 0, 0)),
        ],
        out_specs=pl.BlockSpec(memory_space=pltpu.SMEM),
        out_shape=jax.ShapeDtypeStruct((1, 1), jnp.float32),
    )(cos, partials)


def kernel(datapoints, centroid):
    cos, idx3, best3, dist3 = _pass_a(datapoints, centroid)
    idx = idx3.reshape(_B)
    dist = dist3.reshape(_B)
    best = best3.reshape(_B)
    partials = _sc_hist(idx, dist, best)
    loss = _pass_c(cos, partials)
    return loss.reshape(())


# speedup vs baseline: 1.6572x; 1.0017x over previous
import functools

import jax
import jax.numpy as jnp
from jax import lax
from jax.experimental import pallas as pl
from jax.experimental.pallas import tpu as pltpu
from jax.experimental.pallas import tpu_sc as plsc

_B = 4096
_K = 1024
_D = 256
_SMOOTH = 10.0
_LOG2E = 1.4426950408889634
_LN2 = 0.6931471805599453

_BM = 1024
_GB = _B // _BM

_NC = 2
_NS = 16
_NW = _NC * _NS
_PTS = _B // _NW


def _pass_a_body(dp_ref, cent_ref, cos_ref, idx_ref, best_ref, dist_ref):
    dp = dp_ref[...]
    cent = cent_ref[...]
    num = lax.dot_general(dp, cent, (((1,), (1,)), ((), ())),
                          preferred_element_type=jnp.float32,
                          precision=lax.Precision.DEFAULT)
    dn2 = jnp.sum(dp * dp, axis=1, keepdims=True)
    cn2 = jnp.sum(cent * cent, axis=1)
    inv_dn = 1.0 / jnp.maximum(jnp.sqrt(dn2), 1e-4)
    cn = jnp.maximum(jnp.sqrt(cn2), 1e-4)
    inv_cn = 1.0 / cn
    scaled = num * inv_cn[None, :]
    ms = jnp.max(scaled, axis=1, keepdims=True)
    onehot = (scaled == ms).astype(jnp.bfloat16)
    kio = lax.broadcasted_iota(jnp.int32, (1, _K), 1)
    khi = (kio >> 5).astype(jnp.float32)
    klo = (kio & 31).astype(jnp.float32)
    w3 = jnp.concatenate([khi, klo, cn2[None, :]], axis=0).astype(jnp.bfloat16)
    res = lax.dot_general(onehot, w3, (((1,), (1,)), ((), ())),
                          preferred_element_type=jnp.float32)
    idx_f = res[:, 0] * 32.0 + res[:, 1]
    idx = jnp.minimum(idx_f, float(_K - 1)).astype(jnp.int32)
    cn2_at = res[:, 2]
    cos_ref[...] = (scaled * inv_dn).astype(jnp.bfloat16)
    num_at = ms[:, 0] * jnp.maximum(jnp.sqrt(cn2_at), 1e-4)
    sq = dn2[:, 0] + cn2_at - 2.0 * num_at
    dist = jnp.sqrt(jnp.maximum(sq, 1e-12))
    idx_ref[0, 0, :] = idx
    best_ref[0, 0, :] = ms[:, 0] * inv_dn[:, 0]
    dist_ref[0, 0, :] = dist


def _pass_a(dp, cent):
    return pl.pallas_call(
        _pass_a_body,
        grid=(_GB,),
        in_specs=[
            pl.BlockSpec((_BM, _D), lambda i: (i, 0)),
            pl.BlockSpec((_K, _D), lambda i: (0, 0)),
        ],
        out_specs=[
            pl.BlockSpec((_BM, _K), lambda i: (i, 0)),
            pl.BlockSpec((1, 1, _BM), lambda i: (i, 0, 0)),
            pl.BlockSpec((1, 1, _BM), lambda i: (i, 0, 0)),
            pl.BlockSpec((1, 1, _BM), lambda i: (i, 0, 0)),
        ],
        out_shape=[
            jax.ShapeDtypeStruct((_B, _K), jnp.bfloat16),
            jax.ShapeDtypeStruct((_GB, 1, _BM), jnp.int32),
            jax.ShapeDtypeStruct((_GB, 1, _BM), jnp.float32),
            jax.ShapeDtypeStruct((_GB, 1, _BM), jnp.float32),
        ],
    )(dp, cent)


def _sc_hist(idx, dist, best):
    mesh = plsc.VectorSubcoreMesh(core_axis_name="c", subcore_axis_name="s")

    @functools.partial(
        pl.kernel, mesh=mesh,
        out_type=jax.ShapeDtypeStruct((_NC, 4, _K), jnp.float32),
        scratch_types=[
            pltpu.VMEM((_PTS,), jnp.int32),
            pltpu.VMEM((_PTS,), jnp.float32),
            pltpu.VMEM((_PTS,), jnp.float32),
            pltpu.VMEM((_PTS,), jnp.float32),
            pltpu.VMEM((_K,), jnp.float32),
            pltpu.VMEM_SHARED((_K,), jnp.float32),
            pltpu.VMEM_SHARED((_K,), jnp.float32),
            pltpu.VMEM_SHARED((_K,), jnp.float32),
        ],
    )
    def k(idx_hbm, dist_hbm, best_hbm, out_hbm, idx_v, dist_v, best_v,
          ones_v, zer_v, sh_cnt, sh_sum, sh_best):
        c = lax.axis_index("c")
        s = lax.axis_index("s")
        wid = c * _NS + s
        base = wid * _PTS
        pltpu.sync_copy(idx_hbm.at[pl.ds(base, _PTS)], idx_v)
        pltpu.sync_copy(dist_hbm.at[pl.ds(base, _PTS)], dist_v)
        pltpu.sync_copy(best_hbm.at[pl.ds(base, _PTS)], best_v)
        for i in range(_PTS // 16):
            ones_v[pl.ds(i * 16, 16)] = jnp.ones((16,), jnp.float32)

        @pl.when(s == 0)
        def _zero():
            for i in range(_K // 16):
                zer_v[pl.ds(i * 16, 16)] = jnp.zeros((16,), jnp.float32)
            pltpu.sync_copy(zer_v, sh_cnt)
            pltpu.sync_copy(zer_v, sh_sum)
            pltpu.sync_copy(zer_v, sh_best)
            pltpu.sync_copy(zer_v, out_hbm.at[c, 3])

        plsc.subcore_barrier()
        pltpu.sync_copy(ones_v, sh_cnt.at[idx_v], add=True)
        pltpu.sync_copy(dist_v, sh_sum.at[idx_v], add=True)
        pltpu.sync_copy(best_v, sh_best.at[idx_v], add=True)
        plsc.subcore_barrier()

        @pl.when(s == 0)
        def _emit():
            pltpu.sync_copy(sh_cnt, out_hbm.at[c, 0])
            pltpu.sync_copy(sh_sum, out_hbm.at[c, 1])
            pltpu.sync_copy(sh_best, out_hbm.at[c, 2])

    return k(idx, dist, best)


def _pass_c_body(cos_ref, part_ref, out_ref):
    i = pl.program_id(0)
    cnt = part_ref[0, 0, :] + part_ref[1, 0, :]
    l2 = part_ref[0, 1, :] + part_ref[1, 1, :]
    bs = part_ref[0, 2, :] + part_ref[1, 2, :]
    scnt = jnp.where(cnt == 0.0, 1.0, cnt)
    phi = (l2 / scnt) / jnp.log(scnt + _SMOOTH)
    zmask = jnp.where(cnt == 0.0, -10000.0, 0.0)
    col2 = _LOG2E / (phi + zmask)
    zm2 = zmask * _LOG2E
    smooth2 = zm2[None, :] + cos_ref[...].astype(jnp.float32) * col2[None, :]
    m2 = jnp.max(smooth2, axis=1, keepdims=True)
    e2 = jnp.exp2(smooth2 - m2)
    logz = (m2[:, 0] + jnp.log2(jnp.sum(e2, axis=1))) * _LN2
    part = jnp.sum(logz)

    @pl.when(i == 0)
    def _init():
        picked_sum = jnp.sum(jnp.where(cnt == 0.0, 0.0, bs / phi))
        out_ref[0, 0] = -picked_sum / _B

    out_ref[0, 0] += part / _B


def _pass_c(cos, partials):
    return pl.pallas_call(
        _pass_c_body,
        grid=(_GB,),
        in_specs=[
            pl.BlockSpec((_BM, _K), lambda i: (i, 0)),
            pl.BlockSpec((_NC, 4, _K), lambda i: (0, 0, 0)),
        ],
        out_specs=pl.BlockSpec(memory_space=pltpu.SMEM),
        out_shape=jax.ShapeDtypeStruct((1, 1), jnp.float32),
    )(cos, partials)


def kernel(datapoints, centroid):
    cos, idx3, best3, dist3 = _pass_a(datapoints, centroid)
    idx = idx3.reshape(_B)
    dist = dist3.reshape(_B)
    best = best3.reshape(_B)
    partials = _sc_hist(idx, dist, best)
    loss = _pass_c(cos, partials)
    return loss.reshape(())


# speedup vs baseline: 1.6609x; 1.0023x over previous
import functools

import jax
import jax.numpy as jnp
from jax import lax
from jax.experimental import pallas as pl
from jax.experimental.pallas import tpu as pltpu
from jax.experimental.pallas import tpu_sc as plsc

_B = 4096
_K = 1024
_D = 256
_SMOOTH = 10.0
_LOG2E = 1.4426950408889634
_LN2 = 0.6931471805599453

_BM = 1024
_GB = _B // _BM
_BMA = 2048
_GA = _B // _BMA

_NC = 2
_NS = 16
_NW = _NC * _NS
_PTS = _B // _NW


def _pass_a_body(dp_ref, cent_ref, cos_ref, idx_ref, best_ref, dist_ref):
    dp = dp_ref[...]
    cent = cent_ref[...]
    num = lax.dot_general(dp, cent, (((1,), (1,)), ((), ())),
                          preferred_element_type=jnp.float32,
                          precision=lax.Precision.DEFAULT)
    dn2 = jnp.sum(dp * dp, axis=1, keepdims=True)
    cn2 = jnp.sum(cent * cent, axis=1)
    inv_dn = 1.0 / jnp.maximum(jnp.sqrt(dn2), 1e-4)
    cn = jnp.maximum(jnp.sqrt(cn2), 1e-4)
    inv_cn = 1.0 / cn
    scaled = num * inv_cn[None, :]
    ms = jnp.max(scaled, axis=1, keepdims=True)
    onehot = (scaled == ms).astype(jnp.bfloat16)
    kio = lax.broadcasted_iota(jnp.int32, (1, _K), 1)
    khi = (kio >> 5).astype(jnp.float32)
    klo = (kio & 31).astype(jnp.float32)
    w3 = jnp.concatenate([khi, klo, cn2[None, :]], axis=0).astype(jnp.bfloat16)
    res = lax.dot_general(onehot, w3, (((1,), (1,)), ((), ())),
                          preferred_element_type=jnp.float32)
    idx_f = res[:, 0] * 32.0 + res[:, 1]
    idx = jnp.minimum(idx_f, float(_K - 1)).astype(jnp.int32)
    cn2_at = res[:, 2]
    cos_ref[...] = (scaled * inv_dn).astype(jnp.bfloat16)
    num_at = ms[:, 0] * jnp.maximum(jnp.sqrt(cn2_at), 1e-4)
    sq = dn2[:, 0] + cn2_at - 2.0 * num_at
    dist = jnp.sqrt(jnp.maximum(sq, 1e-12))
    idx_ref[0, 0, :] = idx
    best_ref[0, 0, :] = ms[:, 0] * inv_dn[:, 0]
    dist_ref[0, 0, :] = dist


def _pass_a(dp, cent):
    return pl.pallas_call(
        _pass_a_body,
        grid=(_GA,),
        in_specs=[
            pl.BlockSpec((_BMA, _D), lambda i: (i, 0)),
            pl.BlockSpec((_K, _D), lambda i: (0, 0)),
        ],
        out_specs=[
            pl.BlockSpec((_BMA, _K), lambda i: (i, 0)),
            pl.BlockSpec((1, 1, _BMA), lambda i: (i, 0, 0)),
            pl.BlockSpec((1, 1, _BMA), lambda i: (i, 0, 0)),
            pl.BlockSpec((1, 1, _BMA), lambda i: (i, 0, 0)),
        ],
        out_shape=[
            jax.ShapeDtypeStruct((_B, _K), jnp.bfloat16),
            jax.ShapeDtypeStruct((_GA, 1, _BMA), jnp.int32),
            jax.ShapeDtypeStruct((_GA, 1, _BMA), jnp.float32),
            jax.ShapeDtypeStruct((_GA, 1, _BMA), jnp.float32),
        ],
    )(dp, cent)


def _sc_hist(idx, dist, best):
    mesh = plsc.VectorSubcoreMesh(core_axis_name="c", subcore_axis_name="s")

    @functools.partial(
        pl.kernel, mesh=mesh,
        out_type=jax.ShapeDtypeStruct((_NC, 4, _K), jnp.float32),
        scratch_types=[
            pltpu.VMEM((_PTS,), jnp.int32),
            pltpu.VMEM((_PTS,), jnp.float32),
            pltpu.VMEM((_PTS,), jnp.float32),
            pltpu.VMEM((_PTS,), jnp.float32),
            pltpu.VMEM((_K,), jnp.float32),
            pltpu.VMEM_SHARED((_K,), jnp.float32),
            pltpu.VMEM_SHARED((_K,), jnp.float32),
            pltpu.VMEM_SHARED((_K,), jnp.float32),
        ],
    )
    def k(idx_hbm, dist_hbm, best_hbm, out_hbm, idx_v, dist_v, best_v,
          ones_v, zer_v, sh_cnt, sh_sum, sh_best):
        c = lax.axis_index("c")
        s = lax.axis_index("s")
        wid = c * _NS + s
        base = wid * _PTS
        pltpu.sync_copy(idx_hbm.at[pl.ds(base, _PTS)], idx_v)
        pltpu.sync_copy(dist_hbm.at[pl.ds(base, _PTS)], dist_v)
        pltpu.sync_copy(best_hbm.at[pl.ds(base, _PTS)], best_v)
        for i in range(_PTS // 16):
            ones_v[pl.ds(i * 16, 16)] = jnp.ones((16,), jnp.float32)

        @pl.when(s == 0)
        def _zero():
            for i in range(_K // 16):
                zer_v[pl.ds(i * 16, 16)] = jnp.zeros((16,), jnp.float32)
            pltpu.sync_copy(zer_v, sh_cnt)
            pltpu.sync_copy(zer_v, sh_sum)
            pltpu.sync_copy(zer_v, sh_best)
            pltpu.sync_copy(zer_v, out_hbm.at[c, 3])

        plsc.subcore_barrier()
        pltpu.sync_copy(ones_v, sh_cnt.at[idx_v], add=True)
        pltpu.sync_copy(dist_v, sh_sum.at[idx_v], add=True)
        pltpu.sync_copy(best_v, sh_best.at[idx_v], add=True)
        plsc.subcore_barrier()

        @pl.when(s == 0)
        def _emit():
            pltpu.sync_copy(sh_cnt, out_hbm.at[c, 0])
            pltpu.sync_copy(sh_sum, out_hbm.at[c, 1])
            pltpu.sync_copy(sh_best, out_hbm.at[c, 2])

    return k(idx, dist, best)


def _pass_c_body(cos_ref, part_ref, out_ref):
    i = pl.program_id(0)
    cnt = part_ref[0, 0, :] + part_ref[1, 0, :]
    l2 = part_ref[0, 1, :] + part_ref[1, 1, :]
    bs = part_ref[0, 2, :] + part_ref[1, 2, :]
    scnt = jnp.where(cnt == 0.0, 1.0, cnt)
    phi = (l2 / scnt) / jnp.log(scnt + _SMOOTH)
    zmask = jnp.where(cnt == 0.0, -10000.0, 0.0)
    col2 = _LOG2E / (phi + zmask)
    zm2 = zmask * _LOG2E
    smooth2 = zm2[None, :] + cos_ref[...].astype(jnp.float32) * col2[None, :]
    m2 = jnp.max(smooth2, axis=1, keepdims=True)
    e2 = jnp.exp2(smooth2 - m2)
    logz = (m2[:, 0] + jnp.log2(jnp.sum(e2, axis=1))) * _LN2
    part = jnp.sum(logz)

    @pl.when(i == 0)
    def _init():
        picked_sum = jnp.sum(jnp.where(cnt == 0.0, 0.0, bs / phi))
        out_ref[0, 0] = -picked_sum / _B

    out_ref[0, 0] += part / _B


def _pass_c(cos, partials):
    return pl.pallas_call(
        _pass_c_body,
        grid=(_GB,),
        in_specs=[
            pl.BlockSpec((_BM, _K), lambda i: (i, 0)),
            pl.BlockSpec((_NC, 4, _K), lambda i: (0, 0, 0)),
        ],
        out_specs=pl.BlockSpec(memory_space=pltpu.SMEM),
        out_shape=jax.ShapeDtypeStruct((1, 1), jnp.float32),
    )(cos, partials)


def kernel(datapoints, centroid):
    cos, idx3, best3, dist3 = _pass_a(datapoints, centroid)
    idx = idx3.reshape(_B)
    dist = dist3.reshape(_B)
    best = best3.reshape(_B)
    partials = _sc_hist(idx, dist, best)
    loss = _pass_c(cos, partials)
    return loss.reshape(())
